# baseline (speedup 1.0000x reference)
import math

import jax
import jax.numpy as jnp
from jax import lax
from jax.experimental import pallas as pl
from jax.experimental.pallas import tpu as pltpu
from jax.experimental.pallas import tpu_sc as plsc

D_MODEL = 64
SCALE = math.sqrt(D_MODEL)

NC = 2
NS = 16
NW = NC * NS
CH = 128
NBUF = 2


def _make_kernel(n_idx):
    assert n_idx % (NW * CH) == 0
    chunks_per_w = n_idx // (NW * CH)
    mesh = plsc.VectorSubcoreMesh(core_axis_name="c", subcore_axis_name="s",
                                  num_cores=NC, num_subcores=NS)

    def emb_kernel(idx_hbm, table_hbm, out_hbm, idx_v, phys_v, rows_v, sta_v,
                   gsem, wsem):
        wid = lax.axis_index("s") * NC + lax.axis_index("c")
        pltpu.sync_copy(idx_hbm.at[pl.ds(wid * chunks_per_w, chunks_per_w)],
                        idx_v)
        out_base = wid * chunks_per_w * CH

        def prep_and_gather(c, b):
            for k in range(CH // 16):
                phys_v[b, pl.ds(k * 16, 16)] = (
                    idx_v[c, pl.ds(k * 16, 16)] >> 1)
            pltpu.async_copy(table_hbm.at[phys_v.at[b]], rows_v.at[b],
                             gsem.at[b])

        prep_and_gather(0, 0)

        def body(c, _):
            b = lax.rem(c, 2)
            for bs in range(NBUF):
                @pl.when(b == bs)
                def _():
                    nb = 1 - bs

                    @pl.when(c + 1 < chunks_per_w)
                    def _():
                        prep_and_gather(c + 1, nb)

                    pltpu.make_async_copy(table_hbm.at[phys_v.at[bs]],
                                          rows_v.at[bs], gsem.at[bs]).wait()

                    @pl.when(c >= 2)
                    def _():
                        old = out_base + (c - 2) * CH
                        pltpu.make_async_copy(
                            sta_v.at[bs],
                            out_hbm.at[pl.ds(pl.multiple_of(old, 8), CH)],
                            wsem.at[bs]).wait()

                    @plsc.parallel_loop(0, CH // 16)
                    def select_group(g):
                        parv = idx_v[c, pl.ds(g * 16, 16)] & 1
                        for jj in range(16):
                            off = parv[jj] * 64
                            j = g * 16 + jj
                            for s in range(4):
                                seg = rows_v[bs, j, pl.ds(off + s * 16, 16)]
                                sta_v[bs, j, pl.ds(s * 16, 16)] = seg * SCALE

                    dst = pl.multiple_of(out_base + c * CH, 8)
                    pltpu.async_copy(sta_v.at[bs],
                                     out_hbm.at[pl.ds(dst, CH)], wsem.at[bs])

            return 0

        lax.fori_loop(0, chunks_per_w, body, 0)

        for c in (chunks_per_w - 2, chunks_per_w - 1):
            b = c % NBUF
            pltpu.make_async_copy(
                sta_v.at[b],
                out_hbm.at[pl.ds(out_base + c * CH, CH)],
                wsem.at[b]).wait()

    return pl.kernel(
        emb_kernel,
        out_type=jax.ShapeDtypeStruct((n_idx, D_MODEL), jnp.float32),
        mesh=mesh,
        compiler_params=pltpu.CompilerParams(use_tc_tiling_on_sc=True),
        scratch_types=[
            pltpu.VMEM((chunks_per_w, CH), jnp.int32),
            pltpu.VMEM((NBUF, CH), jnp.int32),
            pltpu.VMEM((NBUF, CH, 2 * D_MODEL), jnp.float32),
            pltpu.VMEM((NBUF, CH, D_MODEL), jnp.float32),
            pltpu.SemaphoreType.DMA((NBUF,)),
            pltpu.SemaphoreType.DMA((NBUF,)),
        ],
    )


def kernel(x, lut):
    n_idx = x.shape[0] * x.shape[1]
    idx2d = x.reshape(n_idx // CH, CH)
    lutp = jnp.concatenate([lut[0::2], lut[1::2]], axis=1)
    out = _make_kernel(n_idx)(idx2d, lutp)
    return out.reshape(x.shape[0], x.shape[1], D_MODEL)


# speedup vs baseline: 8.1632x; 8.1632x over previous
import math

import jax
import jax.numpy as jnp
from jax import lax
from jax.experimental import pallas as pl
from jax.experimental.pallas import tpu as pltpu
from jax.experimental.pallas import tpu_sc as plsc

D_MODEL = 64
SCALE = math.sqrt(D_MODEL)

NC = 2
NS = 16
NW = NC * NS
CH = 128
NBUF = 2


def _make_kernel(n_idx):
    assert n_idx % (NW * CH) == 0
    chunks_per_w = n_idx // (NW * CH)
    mesh = plsc.VectorSubcoreMesh(core_axis_name="c", subcore_axis_name="s",
                                  num_cores=NC, num_subcores=NS)

    def emb_kernel(idx_hbm, table_hbm, out_hbm, idx_v, phys_v, rows_v, sta_v,
                   gsem, wsem):
        wid = lax.axis_index("s") * NC + lax.axis_index("c")
        pltpu.sync_copy(idx_hbm.at[pl.ds(wid * chunks_per_w, chunks_per_w)],
                        idx_v)
        out_base = wid * chunks_per_w * CH

        def prep_and_gather(c, b):
            for k in range(CH // 16):
                phys_v[b, pl.ds(k * 16, 16)] = (
                    idx_v[c, pl.ds(k * 16, 16)] >> 1)
            pltpu.async_copy(table_hbm.at[phys_v.at[b]], rows_v.at[b],
                             gsem.at[b])

        prep_and_gather(0, 0)

        def body(c, _):
            b = lax.rem(c, 2)
            for bs in range(NBUF):
                @pl.when(b == bs)
                def _():
                    nb = 1 - bs

                    @pl.when(c + 1 < chunks_per_w)
                    def _():
                        prep_and_gather(c + 1, nb)

                    pltpu.make_async_copy(table_hbm.at[phys_v.at[bs]],
                                          rows_v.at[bs], gsem.at[bs]).wait()

                    @pl.when(c >= 2)
                    def _():
                        old = out_base + (c - 2) * CH
                        pltpu.make_async_copy(
                            sta_v.at[bs],
                            out_hbm.at[pl.ds(pl.multiple_of(old, 8), CH)],
                            wsem.at[bs]).wait()

                    @plsc.parallel_loop(0, CH // 16)
                    def select_group(g):
                        parv = idx_v[c, pl.ds(g * 16, 16)] & 1
                        for jj in range(16):
                            off = parv[jj] * 64
                            j = g * 16 + jj
                            for s in range(4):
                                seg = rows_v[bs, j, pl.ds(off + s * 16, 16)]
                                sta_v[bs, j, pl.ds(s * 16, 16)] = seg * SCALE

                    dst = pl.multiple_of(out_base + c * CH, 8)
                    pltpu.async_copy(sta_v.at[bs],
                                     out_hbm.at[pl.ds(dst, CH)], wsem.at[bs])

            return 0

        lax.fori_loop(0, chunks_per_w, body, 0)

        for c in (chunks_per_w - 2, chunks_per_w - 1):
            b = c % NBUF
            pltpu.make_async_copy(
                sta_v.at[b],
                out_hbm.at[pl.ds(out_base + c * CH, CH)],
                wsem.at[b]).wait()

    return pl.kernel(
        emb_kernel,
        out_type=jax.ShapeDtypeStruct((n_idx, D_MODEL), jnp.float32),
        mesh=mesh,
        compiler_params=pltpu.CompilerParams(use_tc_tiling_on_sc=True),
        scratch_types=[
            pltpu.VMEM((chunks_per_w, CH), jnp.int32),
            pltpu.VMEM((NBUF, CH), jnp.int32),
            pltpu.VMEM((NBUF, CH, 2 * D_MODEL), jnp.float32),
            pltpu.VMEM((NBUF, CH, D_MODEL), jnp.float32),
            pltpu.SemaphoreType.DMA((NBUF,)),
            pltpu.SemaphoreType.DMA((NBUF,)),
        ],
    )


def kernel(x, lut):
    n_idx = x.shape[0] * x.shape[1]
    idx2d = x.reshape(n_idx // CH, CH)
    lutp = lut.reshape(lut.shape[0] // 2, 2 * D_MODEL)
    out = _make_kernel(n_idx)(idx2d, lutp)
    return out.reshape(x.shape[0], x.shape[1], D_MODEL)
